# baseline (speedup 1.0000x reference)
import functools

import jax
import jax.numpy as jnp
import numpy as np
from jax import lax
from jax.experimental import pallas as pl
from jax.experimental.pallas import tpu as pltpu
from jax.experimental.pallas import tpu_sc as plsc

B = 16384
D = 256
TOK = 4
CD = 64
NC = 2
NS = 16
NW = NC * NS
ROWS_PER_W = B // NW
CHUNK = 128
NCHUNKS = ROWS_PER_W // CHUNK

ROWS_TC = 1024
GRID_TC = B // ROWS_TC


def _sc_gather_body(ids_hbm, embed_hbm, out_hbm,
                    idx0, idx1, buf0, buf1, sem0, sem1):
    wid = lax.axis_index("s") * NC + lax.axis_index("c")
    base = wid * ROWS_PER_W
    idx = (idx0, idx1)
    buf = (buf0, buf1)
    sem = (sem0, sem1)

    pltpu.sync_copy(ids_hbm.at[pl.ds(base, CHUNK)], idx0)
    cp = pltpu.async_copy(embed_hbm.at[idx0], buf0, sem0)
    copies = [cp]
    for c in range(NCHUNKS):
        nxt = (c + 1) % 2
        if c + 1 < NCHUNKS:
            pltpu.sync_copy(ids_hbm.at[pl.ds(base + (c + 1) * CHUNK, CHUNK)],
                            idx[nxt])
            copies.append(pltpu.async_copy(embed_hbm.at[idx[nxt]], buf[nxt],
                                           sem[nxt]))
        copies[c].wait()
        pltpu.sync_copy(buf[c % 2], out_hbm.at[pl.ds(base + c * CHUNK, CHUNK)])


_sc_gather = functools.partial(
    pl.kernel,
    out_type=jax.ShapeDtypeStruct((B, D), jnp.float32),
    mesh=plsc.VectorSubcoreMesh(core_axis_name="c", subcore_axis_name="s"),
    scratch_types=[
        pltpu.VMEM((CHUNK,), jnp.int32),
        pltpu.VMEM((CHUNK,), jnp.int32),
        pltpu.VMEM((CHUNK, D), jnp.float32),
        pltpu.VMEM((CHUNK, D), jnp.float32),
        pltpu.SemaphoreType.DMA,
        pltpu.SemaphoreType.DMA,
    ],
)(_sc_gather_body)


def _ln_body(x_ref, s_ref, w_ref, b_ref, o_ref):
    x = x_ref[...]
    s = s_ref[...]
    m = jax.lax.dot(x, s, preferred_element_type=jnp.float32)
    q = jax.lax.dot(x * x, s, preferred_element_type=jnp.float32)
    r = lax.rsqrt(q - m * m + jnp.float32(1e-5))
    y = (x - m) * (r * w_ref[...]) + b_ref[...]
    o_ref[...] = y.T


_ln_call = pl.pallas_call(
    _ln_body,
    grid=(GRID_TC,),
    in_specs=[
        pl.BlockSpec((ROWS_TC, D), lambda i: (i, 0)),
        pl.BlockSpec((D, D), lambda i: (0, 0)),
        pl.BlockSpec((1, D), lambda i: (0, 0)),
        pl.BlockSpec((1, D), lambda i: (0, 0)),
    ],
    out_specs=pl.BlockSpec((D, ROWS_TC), lambda i: (0, i)),
    out_shape=jax.ShapeDtypeStruct((D, B), jnp.float32),
)


_SEG = np.kron(np.eye(TOK, dtype=np.float32),
               np.full((CD, CD), 1.0 / CD, dtype=np.float32))


def kernel(class_ids, embed, ln_weight, ln_bias):
    ids = class_ids.astype(jnp.int32)
    seg = jnp.asarray(_SEG)
    w4 = jnp.tile(ln_weight, TOK).reshape(1, D)
    b4 = jnp.tile(ln_bias, TOK).reshape(1, D)
    g = _sc_gather(ids, embed)
    out_t = _ln_call(g, seg, w4, b4)
    return out_t.T.reshape(B, TOK, CD)


# speedup vs baseline: 1.1510x; 1.1510x over previous
import functools

import jax
import jax.numpy as jnp
import numpy as np
from jax import lax
from jax.experimental import pallas as pl
from jax.experimental.pallas import tpu as pltpu
from jax.experimental.pallas import tpu_sc as plsc

B = 16384
D = 256
TOK = 4
CD = 64
NC = 2
NS = 16
NW = NC * NS
ROWS_PER_W = B // NW
CHUNK = 128
NCHUNKS = ROWS_PER_W // CHUNK

ROWS_TC = 4096
GRID_TC = B // ROWS_TC


def _sc_gather_body(ids_hbm, embed_hbm, out_hbm,
                    idx0, idx1, buf0, buf1, sem0, sem1):
    wid = lax.axis_index("s") * NC + lax.axis_index("c")
    base = wid * ROWS_PER_W
    idx = (idx0, idx1)
    buf = (buf0, buf1)
    sem = (sem0, sem1)

    pltpu.sync_copy(ids_hbm.at[pl.ds(base, CHUNK)], idx0)
    cp = pltpu.async_copy(embed_hbm.at[idx0], buf0, sem0)
    copies = [cp]
    for c in range(NCHUNKS):
        nxt = (c + 1) % 2
        if c + 1 < NCHUNKS:
            pltpu.sync_copy(ids_hbm.at[pl.ds(base + (c + 1) * CHUNK, CHUNK)],
                            idx[nxt])
            copies.append(pltpu.async_copy(embed_hbm.at[idx[nxt]], buf[nxt],
                                           sem[nxt]))
        copies[c].wait()
        pltpu.sync_copy(buf[c % 2], out_hbm.at[pl.ds(base + c * CHUNK, CHUNK)])


_sc_gather = functools.partial(
    pl.kernel,
    out_type=jax.ShapeDtypeStruct((B, D), jnp.float32),
    mesh=plsc.VectorSubcoreMesh(core_axis_name="c", subcore_axis_name="s"),
    scratch_types=[
        pltpu.VMEM((CHUNK,), jnp.int32),
        pltpu.VMEM((CHUNK,), jnp.int32),
        pltpu.VMEM((CHUNK, D), jnp.float32),
        pltpu.VMEM((CHUNK, D), jnp.float32),
        pltpu.SemaphoreType.DMA,
        pltpu.SemaphoreType.DMA,
    ],
)(_sc_gather_body)


def _ln_body(x_ref, s_ref, w_ref, b_ref, o_ref):
    x = x_ref[...]
    s = s_ref[...]
    m = jax.lax.dot(x, s, preferred_element_type=jnp.float32)
    q = jax.lax.dot(x * x, s, preferred_element_type=jnp.float32)
    r = lax.rsqrt(q - m * m + jnp.float32(1e-5))
    y = (x - m) * (r * w_ref[...]) + b_ref[...]
    o_ref[...] = y.T


_ln_call = pl.pallas_call(
    _ln_body,
    grid=(GRID_TC,),
    in_specs=[
        pl.BlockSpec((ROWS_TC, D), lambda i: (i, 0)),
        pl.BlockSpec((D, D), lambda i: (0, 0)),
        pl.BlockSpec((1, D), lambda i: (0, 0)),
        pl.BlockSpec((1, D), lambda i: (0, 0)),
    ],
    out_specs=pl.BlockSpec((D, ROWS_TC), lambda i: (0, i)),
    out_shape=jax.ShapeDtypeStruct((D, B), jnp.float32),
)


_SEG = np.kron(np.eye(TOK, dtype=np.float32),
               np.full((CD, CD), 1.0 / CD, dtype=np.float32))


def kernel(class_ids, embed, ln_weight, ln_bias):
    ids = class_ids.astype(jnp.int32)
    seg = jnp.asarray(_SEG)
    w4 = jnp.tile(ln_weight, TOK).reshape(1, D)
    b4 = jnp.tile(ln_bias, TOK).reshape(1, D)
    g = _sc_gather(ids, embed)
    out_t = _ln_call(g, seg, w4, b4)
    return out_t.T.reshape(B, TOK, CD)


# speedup vs baseline: 1.1754x; 1.0212x over previous
import functools

import jax
import jax.numpy as jnp
import numpy as np
from jax import lax
from jax.experimental import pallas as pl
from jax.experimental.pallas import tpu as pltpu
from jax.experimental.pallas import tpu_sc as plsc

B = 16384
D = 256
TOK = 4
CD = 64
NC = 2
NS = 16
NW = NC * NS
ROWS_PER_W = B // NW
CHUNK = 128
NCHUNKS = ROWS_PER_W // CHUNK

ROWS_TC = 8192
GRID_TC = B // ROWS_TC


def _sc_gather_body(ids_hbm, embed_hbm, out_hbm,
                    idx0, idx1, buf0, buf1, sem0, sem1):
    wid = lax.axis_index("s") * NC + lax.axis_index("c")
    base = wid * ROWS_PER_W
    idx = (idx0, idx1)
    buf = (buf0, buf1)
    sem = (sem0, sem1)

    pltpu.sync_copy(ids_hbm.at[pl.ds(base, CHUNK)], idx0)
    cp = pltpu.async_copy(embed_hbm.at[idx0], buf0, sem0)
    copies = [cp]
    for c in range(NCHUNKS):
        nxt = (c + 1) % 2
        if c + 1 < NCHUNKS:
            pltpu.sync_copy(ids_hbm.at[pl.ds(base + (c + 1) * CHUNK, CHUNK)],
                            idx[nxt])
            copies.append(pltpu.async_copy(embed_hbm.at[idx[nxt]], buf[nxt],
                                           sem[nxt]))
        copies[c].wait()
        pltpu.sync_copy(buf[c % 2], out_hbm.at[pl.ds(base + c * CHUNK, CHUNK)])


_sc_gather = functools.partial(
    pl.kernel,
    out_type=jax.ShapeDtypeStruct((B, D), jnp.float32),
    mesh=plsc.VectorSubcoreMesh(core_axis_name="c", subcore_axis_name="s"),
    scratch_types=[
        pltpu.VMEM((CHUNK,), jnp.int32),
        pltpu.VMEM((CHUNK,), jnp.int32),
        pltpu.VMEM((CHUNK, D), jnp.float32),
        pltpu.VMEM((CHUNK, D), jnp.float32),
        pltpu.SemaphoreType.DMA,
        pltpu.SemaphoreType.DMA,
    ],
)(_sc_gather_body)


def _ln_body(x_ref, s_ref, w_ref, b_ref, o_ref):
    x = x_ref[...]
    s = s_ref[...]
    m = jax.lax.dot(x, s, preferred_element_type=jnp.float32)
    q = jax.lax.dot(x * x, s, preferred_element_type=jnp.float32)
    r = lax.rsqrt(q - m * m + jnp.float32(1e-5))
    y = (x - m) * (r * w_ref[...]) + b_ref[...]
    o_ref[...] = y.T


_ln_call = pl.pallas_call(
    _ln_body,
    grid=(GRID_TC,),
    in_specs=[
        pl.BlockSpec((ROWS_TC, D), lambda i: (i, 0)),
        pl.BlockSpec((D, D), lambda i: (0, 0)),
        pl.BlockSpec((1, D), lambda i: (0, 0)),
        pl.BlockSpec((1, D), lambda i: (0, 0)),
    ],
    out_specs=pl.BlockSpec((D, ROWS_TC), lambda i: (0, i)),
    out_shape=jax.ShapeDtypeStruct((D, B), jnp.float32),
)


_SEG = np.kron(np.eye(TOK, dtype=np.float32),
               np.full((CD, CD), 1.0 / CD, dtype=np.float32))


def kernel(class_ids, embed, ln_weight, ln_bias):
    ids = class_ids.astype(jnp.int32)
    seg = jnp.asarray(_SEG)
    w4 = jnp.tile(ln_weight, TOK).reshape(1, D)
    b4 = jnp.tile(ln_bias, TOK).reshape(1, D)
    g = _sc_gather(ids, embed)
    out_t = _ln_call(g, seg, w4, b4)
    return out_t.T.reshape(B, TOK, CD)
